# baseline (speedup 1.0000x reference)
import functools
import math

import jax
import jax.numpy as jnp
from jax import lax
from jax.experimental import pallas as pl
from jax.experimental.pallas import tpu as pltpu

_EPS = 1e-5
_BF16 = jnp.bfloat16
_F32 = jnp.float32


def _ln_stats(x):
    mean = jnp.mean(x, axis=-1, keepdims=True)
    meansq = jnp.mean(x * x, axis=-1, keepdims=True)
    var = meansq - mean * mean
    return mean, lax.rsqrt(var + _EPS)


def _ln_f32(x, gamma, beta):
    mean, r = _ln_stats(x)
    return (x - mean) * (r * gamma) + beta


def _block(x2d, w_ref, vec_ref, *, bb, seq, d, f, sig):
    o_wo, o_w1, o_w2 = 3 * d, 4 * d, 4 * d + f
    xb = x2d.astype(_BF16)

    qkv = jnp.dot(xb, w_ref[:, 0:3 * d], preferred_element_type=_F32)
    qkv_b = qkv.astype(_BF16) + vec_ref[0:1, 0:3 * d].astype(_BF16)
    q = qkv_b[:, 0:d].reshape(bb, seq, d)
    k = qkv_b[:, d:2 * d].reshape(bb, seq, d)
    v = qkv_b[:, 2 * d:3 * d].reshape(bb, seq, d)

    s = jnp.einsum("bqd,bkd->bqk", q, k, preferred_element_type=_F32)
    p = jnp.exp(s)
    psum = jnp.sum(p, axis=-1, keepdims=True)
    ctx = jnp.einsum("bqk,bkd->bqd", p.astype(_BF16), v,
                     preferred_element_type=_F32)
    ctx = ctx * pl.reciprocal(psum, approx=True)
    ctx = ctx.reshape(bb * seq, d)

    a_out = jnp.dot(ctx.astype(_BF16), w_ref[:, o_wo:o_wo + d],
                    preferred_element_type=_F32) + vec_ref[1:2, 0:d]
    h = x2d + a_out
    mean1, r1 = _ln_stats(h)
    h = (h - mean1) * (r1 * vec_ref[4:5, 0:d]) + vec_ref[5:6, 0:d]

    ff = jnp.dot(h.astype(_BF16), w_ref[:, o_w1:o_w1 + f],
                 preferred_element_type=_F32)
    ffb = jnp.maximum(ff.astype(_BF16) + vec_ref[2:3, 0:f].astype(_BF16), 0)
    ff = lax.dot_general(ffb, w_ref[:, o_w2:o_w2 + f],
                         (((1,), (1,)), ((), ())),
                         preferred_element_type=_F32)
    out = h + ff + vec_ref[3:4, 0:d]
    mean2, r2 = _ln_stats(out)
    out = (out - mean2) * (r2 * vec_ref[6:7, 0:d]) + vec_ref[7:8, 0:d]
    if sig:
        out = pl.reciprocal(1.0 + jnp.exp(-out), approx=True)
    return out


def _fused_kernel(x_ref, cw_ref, cvec_ref, ew_ref, evec_ref,
                  hw_ref, hvec_ref, o_ref, *, bb, H, W, Fe, Fh):
    hb = bb // 2
    for half in range(2):
        img = slice(half * hb, (half + 1) * hb)
        rows = slice(half * hb * H, (half + 1) * hb * H)
        acc = jnp.dot(x_ref[img, 0].reshape(hb * H, W).astype(_BF16),
                      cw_ref[0], preferred_element_type=_F32)
        acc += jnp.dot(x_ref[img, 1].reshape(hb * H, W).astype(_BF16),
                       cw_ref[1], preferred_element_type=_F32)
        acc += jnp.dot(x_ref[img, 2].reshape(hb * H, W).astype(_BF16),
                       cw_ref[2], preferred_element_type=_F32)
        acc = acc + cvec_ref[0:1, :]
        feat = _ln_f32(acc, cvec_ref[1:2, :], cvec_ref[2:3, :])

        feat = _block(feat, ew_ref, evec_ref,
                      bb=hb, seq=H, d=W, f=Fe, sig=False)
        out = _block(feat, hw_ref, hvec_ref,
                     bb=hb, seq=W, d=H, f=Fh, sig=True)
        o_ref[rows, :] = out.astype(o_ref.dtype)


def _vec_row(v, width):
    v = jnp.asarray(v).reshape(1, -1)
    return jnp.pad(v, ((0, 0), (0, width - v.shape[1])))


def _pack(wq, bq, wk, bk, wv, bv, wo, bo, w1, b1, w2, b2,
          g1, be1, g2, be2, d, f):
    scale = 1.0 / math.sqrt(d)
    w = jnp.concatenate([wq * scale, wk, wv, wo, w1, w2.T], axis=1)
    width = max(3 * d, f)
    bqkv = jnp.concatenate([bq * scale, bk, bv], axis=1)
    vec = jnp.concatenate(
        [_vec_row(bqkv, width), _vec_row(bo, width),
         _vec_row(b1, width), _vec_row(b2, width),
         _vec_row(g1, width), _vec_row(be1, width),
         _vec_row(g2, width), _vec_row(be2, width)], axis=0)
    return w.astype(_BF16), vec.astype(_F32)


def _const2(a):
    zeros = (0,) * a.ndim
    return pl.BlockSpec(a.shape, lambda i, _z=zeros: _z)


def kernel(x, comb_w, comb_b, comb_gamma, comb_beta,
           enc_wq, enc_bq, enc_wk, enc_bk, enc_wv, enc_bv, enc_wo, enc_bo,
           enc_w1, enc_b1, enc_w2, enc_b2, enc_ln1g, enc_ln1b, enc_ln2g,
           enc_ln2b,
           head_wq, head_bq, head_wk, head_bk, head_wv, head_bv, head_wo,
           head_bo, head_w1, head_b1, head_w2, head_b2, head_ln1g, head_ln1b,
           head_ln2g, head_ln2b):
    B, C, H, W = x.shape
    assert C == 3 and H == W and H % 8 == 0
    Fe = enc_w1.shape[1]
    Fh = head_w1.shape[1]

    bb = 32 if B % 32 == 0 else (8 if B % 8 == 0 else 1)
    assert B % bb == 0
    steps = B // bb
    assert steps % 2 == 0

    cw = comb_w.astype(_BF16)
    cvec = jnp.concatenate(
        [comb_b.reshape(1, W), comb_gamma.reshape(1, W),
         comb_beta.reshape(1, W)], axis=0).astype(_F32)
    ew, evec = _pack(enc_wq, enc_bq, enc_wk, enc_bk, enc_wv, enc_bv,
                     enc_wo, enc_bo, enc_w1, enc_b1, enc_w2, enc_b2,
                     enc_ln1g, enc_ln1b, enc_ln2g, enc_ln2b, W, Fe)
    hw, hvec = _pack(head_wq, head_bq, head_wk, head_bk, head_wv, head_bv,
                     head_wo, head_bo, head_w1, head_b1, head_w2, head_b2,
                     head_ln1g, head_ln1b, head_ln2g, head_ln2b, H, Fh)

    def bflops(S, D, F):
        return 4 * 2 * S * D * D + 2 * 2 * S * S * D + 2 * 2 * S * D * F

    flops = B * (2 * H * (3 * W) * W + bflops(H, W, Fe) + bflops(W, H, Fh))
    nbytes = 4 * (x.size + B * H * W) + 2 * (cw.size + ew.size + hw.size)

    out = pl.pallas_call(
        functools.partial(_fused_kernel, bb=bb, H=H, W=W, Fe=Fe, Fh=Fh),
        out_shape=jax.ShapeDtypeStruct((B * W, H), _F32),
        grid_spec=pltpu.PrefetchScalarGridSpec(
            num_scalar_prefetch=0, grid=(steps,),
            in_specs=[pl.BlockSpec((bb, 3, H, W), lambda i: (i, 0, 0, 0)),
                      _const2(cw), _const2(cvec),
                      _const2(ew), _const2(evec),
                      _const2(hw), _const2(hvec)],
            out_specs=pl.BlockSpec((bb * W, H), lambda i: (i, 0))),
        compiler_params=pltpu.CompilerParams(
            dimension_semantics=("parallel",)),
        cost_estimate=pl.CostEstimate(
            flops=flops,
            transcendentals=B * (H * H + W * W + 2 * H + 2 * W),
            bytes_accessed=nbytes),
    )(x, cw, cvec, ew, evec, hw, hvec)

    return out.reshape(B, H, W)


# speedup vs baseline: 1.0088x; 1.0088x over previous
import functools
import math

import jax
import jax.numpy as jnp
from jax import lax
from jax.experimental import pallas as pl
from jax.experimental.pallas import tpu as pltpu

_EPS = 1e-5
_BF16 = jnp.bfloat16
_F32 = jnp.float32


def _ln_stats(x, d):
    s1 = jnp.sum(x, axis=-1, keepdims=True)
    s2 = jnp.sum(x * x, axis=-1, keepdims=True)
    mean = s1 * (1.0 / d)
    return mean, lax.rsqrt(s2 - mean * s1 + (_EPS * d))


def _ln_f32(x, gamma_s, beta, d):
    mean, r = _ln_stats(x, d)
    return (x - mean) * (r * gamma_s) + beta


def _block(x2d, w_ref, vec_ref, *, bb, seq, d, f, sig):
    o_wo, o_w1, o_w2 = 3 * d, 4 * d, 4 * d + f
    xb = x2d.astype(_BF16)

    qkv = jnp.dot(xb, w_ref[:, 0:3 * d], preferred_element_type=_F32)
    qkv_b = qkv.astype(_BF16) + vec_ref[0:1, 0:3 * d].astype(_BF16)
    q = qkv_b[:, 0:d].reshape(bb, seq, d)
    k = qkv_b[:, d:2 * d].reshape(bb, seq, d)
    v = qkv_b[:, 2 * d:3 * d].reshape(bb, seq, d)

    s = jnp.einsum("bqd,bkd->bqk", q, k, preferred_element_type=_F32)
    p = jnp.exp(s)
    psum = jnp.sum(p, axis=-1, keepdims=True)
    ctx = jnp.einsum("bqk,bkd->bqd", p.astype(_BF16), v,
                     preferred_element_type=_F32)
    ctx = ctx * pl.reciprocal(psum, approx=True)
    ctx = ctx.reshape(bb * seq, d)

    a_out = jnp.dot(ctx.astype(_BF16), w_ref[:, o_wo:o_wo + d],
                    preferred_element_type=_F32) + vec_ref[1:2, 0:d]
    h = x2d + a_out
    mean1, r1 = _ln_stats(h, d)
    h = (h - mean1) * (r1 * vec_ref[4:5, 0:d]) + vec_ref[5:6, 0:d]

    ff = jnp.dot(h.astype(_BF16), w_ref[:, o_w1:o_w1 + f],
                 preferred_element_type=_F32)
    ffb = jnp.maximum(ff.astype(_BF16) + vec_ref[2:3, 0:f].astype(_BF16), 0)
    ff = lax.dot_general(ffb, w_ref[:, o_w2:o_w2 + f],
                         (((1,), (1,)), ((), ())),
                         preferred_element_type=_F32)
    out = h + ff + vec_ref[3:4, 0:d]
    mean2, r2 = _ln_stats(out, d)
    out = (out - mean2) * (r2 * vec_ref[6:7, 0:d]) + vec_ref[7:8, 0:d]
    if sig:
        out = pl.reciprocal(1.0 + jnp.exp(-out), approx=True)
    return out


def _fused_kernel(x_ref, cw_ref, cvec_ref, ew_ref, evec_ref,
                  hw_ref, hvec_ref, o_ref, *, bb, H, W, Fe, Fh):
    acc = jnp.dot(x_ref[:, 0].reshape(bb * H, W).astype(_BF16), cw_ref[0],
                  preferred_element_type=_F32)
    acc += jnp.dot(x_ref[:, 1].reshape(bb * H, W).astype(_BF16), cw_ref[1],
                   preferred_element_type=_F32)
    acc += jnp.dot(x_ref[:, 2].reshape(bb * H, W).astype(_BF16), cw_ref[2],
                   preferred_element_type=_F32)
    acc = acc + cvec_ref[0:1, :]
    feat = _ln_f32(acc, cvec_ref[1:2, :], cvec_ref[2:3, :], W)

    feat = _block(feat, ew_ref, evec_ref, bb=bb, seq=H, d=W, f=Fe, sig=False)
    out = _block(feat, hw_ref, hvec_ref, bb=bb, seq=W, d=H, f=Fh, sig=True)
    o_ref[...] = out.astype(o_ref.dtype)


def _vec_row(v, width):
    v = jnp.asarray(v).reshape(1, -1)
    return jnp.pad(v, ((0, 0), (0, width - v.shape[1])))


def _pack(wq, bq, wk, bk, wv, bv, wo, bo, w1, b1, w2, b2,
          g1, be1, g2, be2, d, f):
    scale = 1.0 / math.sqrt(d)
    w = jnp.concatenate([wq * scale, wk, wv, wo, w1, w2.T], axis=1)
    width = max(3 * d, f)
    bqkv = jnp.concatenate([bq * scale, bk, bv], axis=1)
    rd = math.sqrt(d)
    vec = jnp.concatenate(
        [_vec_row(bqkv, width), _vec_row(bo, width),
         _vec_row(b1, width), _vec_row(b2, width),
         _vec_row(g1 * rd, width), _vec_row(be1, width),
         _vec_row(g2 * rd, width), _vec_row(be2, width)], axis=0)
    return w.astype(_BF16), vec.astype(_F32)


def _const2(a):
    zeros = (0,) * a.ndim
    return pl.BlockSpec(a.shape, lambda i, _z=zeros: _z)


def kernel(x, comb_w, comb_b, comb_gamma, comb_beta,
           enc_wq, enc_bq, enc_wk, enc_bk, enc_wv, enc_bv, enc_wo, enc_bo,
           enc_w1, enc_b1, enc_w2, enc_b2, enc_ln1g, enc_ln1b, enc_ln2g,
           enc_ln2b,
           head_wq, head_bq, head_wk, head_bk, head_wv, head_bv, head_wo,
           head_bo, head_w1, head_b1, head_w2, head_b2, head_ln1g, head_ln1b,
           head_ln2g, head_ln2b):
    B, C, H, W = x.shape
    assert C == 3 and H == W and H % 8 == 0
    Fe = enc_w1.shape[1]
    Fh = head_w1.shape[1]

    bb = 32 if B % 32 == 0 else (8 if B % 8 == 0 else 1)
    assert B % bb == 0
    steps = B // bb
    assert steps % 2 == 0

    cw = comb_w.astype(_BF16)
    cvec = jnp.concatenate(
        [comb_b.reshape(1, W), comb_gamma.reshape(1, W) * math.sqrt(W),
         comb_beta.reshape(1, W)], axis=0).astype(_F32)
    ew, evec = _pack(enc_wq, enc_bq, enc_wk, enc_bk, enc_wv, enc_bv,
                     enc_wo, enc_bo, enc_w1, enc_b1, enc_w2, enc_b2,
                     enc_ln1g, enc_ln1b, enc_ln2g, enc_ln2b, W, Fe)
    hw, hvec = _pack(head_wq, head_bq, head_wk, head_bk, head_wv, head_bv,
                     head_wo, head_bo, head_w1, head_b1, head_w2, head_b2,
                     head_ln1g, head_ln1b, head_ln2g, head_ln2b, H, Fh)

    def bflops(S, D, F):
        return 4 * 2 * S * D * D + 2 * 2 * S * S * D + 2 * 2 * S * D * F

    flops = B * (2 * H * (3 * W) * W + bflops(H, W, Fe) + bflops(W, H, Fh))
    nbytes = 4 * (x.size + B * H * W) + 2 * (cw.size + ew.size + hw.size)

    out = pl.pallas_call(
        functools.partial(_fused_kernel, bb=bb, H=H, W=W, Fe=Fe, Fh=Fh),
        out_shape=jax.ShapeDtypeStruct((B * W, H), _F32),
        grid_spec=pltpu.PrefetchScalarGridSpec(
            num_scalar_prefetch=0, grid=(steps,),
            in_specs=[pl.BlockSpec((bb, 3, H, W), lambda i: (i, 0, 0, 0)),
                      _const2(cw), _const2(cvec),
                      _const2(ew), _const2(evec),
                      _const2(hw), _const2(hvec)],
            out_specs=pl.BlockSpec((bb * W, H), lambda i: (i, 0))),
        compiler_params=pltpu.CompilerParams(
            dimension_semantics=("parallel",)),
        cost_estimate=pl.CostEstimate(
            flops=flops,
            transcendentals=B * (H * H + W * W + 2 * H + 2 * W),
            bytes_accessed=nbytes),
    )(x, cw, cvec, ew, evec, hw, hvec)

    return out.reshape(B, H, W)
